# baseline (device time: 98667 ns/iter reference)
import jax
import jax.numpy as jnp
from jax import lax
from jax.experimental import pallas as pl
from jax.experimental.pallas import tpu as pltpu

N_DEV = 16
K_SUB = 2


def kernel(x, w_mat):
    m, k = x.shape
    _, n = w_mat.shape
    m_per = m // N_DEV
    nh = n // 2
    m_sub = m_per // K_SUB

    def body(x_ref, w_ref, out_ref,
             fsend_ref, bsend_ref, frecv_ref, brecv_ref,
             fsend_sems, bsend_sems, frecv_sems, brecv_sems):
        p = lax.axis_index("i")
        right = lax.rem(p + 1, N_DEV)
        left = lax.rem(p - 1 + N_DEV, N_DEV)

        def sub_dot(c, ks, lo):
            xs = x_ref[pl.ds(c * m_per + ks * m_sub, m_sub), :]
            return jax.lax.dot_general(
                xs, w_ref[:, pl.ds(lo, nh)],
                (((1,), (0,)), ((), ())),
                preferred_element_type=jnp.float32,
            )

        def make_rdma(s, ks, fwd):
            slot = s % 2
            send_ref = fsend_ref if fwd else bsend_ref
            recv_ref = frecv_ref if fwd else brecv_ref
            send_sems = fsend_sems if fwd else bsend_sems
            recv_sems = frecv_sems if fwd else brecv_sems
            return pltpu.make_async_remote_copy(
                src_ref=send_ref.at[slot, pl.ds(ks * m_sub, m_sub), :],
                dst_ref=recv_ref.at[s, pl.ds(ks * m_sub, m_sub), :],
                send_sem=send_sems.at[s, ks],
                recv_sem=recv_sems.at[s, ks],
                device_id=(right if fwd else left,),
                device_id_type=pl.DeviceIdType.MESH,
            )

        rows = lambda ks: pl.ds(ks * m_sub, m_sub)

        fc0 = lax.rem(p - 1 + N_DEV, N_DEV)
        bc0 = lax.rem(p + 1, N_DEV)
        for ks in range(K_SUB):
            fsend_ref[0, rows(ks), :] = sub_dot(fc0, ks, 0).astype(jnp.bfloat16)
            bsend_ref[0, rows(ks), :] = sub_dot(bc0, ks, nh).astype(jnp.bfloat16)

        barrier_sem = pltpu.get_barrier_semaphore()
        for nbr in (left, right):
            pl.semaphore_signal(
                barrier_sem, inc=1,
                device_id=(nbr,), device_id_type=pl.DeviceIdType.MESH,
            )
        pl.semaphore_wait(barrier_sem, 2)

        for ks in range(K_SUB):
            make_rdma(0, ks, True).start()
            make_rdma(0, ks, False).start()

        fc1 = lax.rem(p - 2 + N_DEV, N_DEV)
        bc1 = lax.rem(p + 2, N_DEV)
        fdots = [sub_dot(fc1, ks, 0) for ks in range(K_SUB)]
        bdots = [sub_dot(bc1, ks, nh) for ks in range(K_SUB)]

        for s in range(1, N_DEV - 1):
            slot = s % 2
            nfc = lax.rem(p - s - 2 + 2 * N_DEV, N_DEV) if s < N_DEV - 2 else p
            nbc = lax.rem(p + s + 2, N_DEV) if s < N_DEV - 2 else p
            nfdots = [sub_dot(nfc, ks, 0) for ks in range(K_SUB)]
            nbdots = [sub_dot(nbc, ks, nh) for ks in range(K_SUB)]
            for ks in range(K_SUB):
                if s >= 2:
                    make_rdma(s - 2, ks, True).wait_send()
                    make_rdma(s - 2, ks, False).wait_send()
                make_rdma(s - 1, ks, True).wait_recv()
                fsend_ref[slot, rows(ks), :] = (
                    frecv_ref[s - 1, rows(ks), :].astype(jnp.float32) + fdots[ks]
                ).astype(jnp.bfloat16)
                make_rdma(s, ks, True).start()

                make_rdma(s - 1, ks, False).wait_recv()
                bsend_ref[slot, rows(ks), :] = (
                    brecv_ref[s - 1, rows(ks), :].astype(jnp.float32) + bdots[ks]
                ).astype(jnp.bfloat16)
                make_rdma(s, ks, False).start()
            fdots, bdots = nfdots, nbdots

        for ks in range(K_SUB):
            make_rdma(N_DEV - 2, ks, True).wait_recv()
            out_ref[rows(ks), pl.ds(0, nh)] = (
                frecv_ref[N_DEV - 2, rows(ks), :].astype(jnp.float32) + fdots[ks]
            )
            make_rdma(N_DEV - 2, ks, False).wait_recv()
            out_ref[rows(ks), pl.ds(nh, nh)] = (
                brecv_ref[N_DEV - 2, rows(ks), :].astype(jnp.float32) + bdots[ks]
            )

        for s in (N_DEV - 3, N_DEV - 2):
            for ks in range(K_SUB):
                make_rdma(s, ks, True).wait_send()
                make_rdma(s, ks, False).wait_send()

    return pl.pallas_call(
        body,
        out_shape=jax.ShapeDtypeStruct((m_per, n), jnp.float32),
        in_specs=[
            pl.BlockSpec(memory_space=pltpu.VMEM),
            pl.BlockSpec(memory_space=pltpu.VMEM),
        ],
        out_specs=pl.BlockSpec(memory_space=pltpu.VMEM),
        scratch_shapes=[
            pltpu.VMEM((2, m_per, nh), jnp.bfloat16),
            pltpu.VMEM((2, m_per, nh), jnp.bfloat16),
            pltpu.VMEM((N_DEV - 1, m_per, nh), jnp.bfloat16),
            pltpu.VMEM((N_DEV - 1, m_per, nh), jnp.bfloat16),
            pltpu.SemaphoreType.DMA((N_DEV - 1, K_SUB)),
            pltpu.SemaphoreType.DMA((N_DEV - 1, K_SUB)),
            pltpu.SemaphoreType.DMA((N_DEV - 1, K_SUB)),
            pltpu.SemaphoreType.DMA((N_DEV - 1, K_SUB)),
        ],
        compiler_params=pltpu.CompilerParams(collective_id=0),
    )(x, w_mat)


# device time: 98371 ns/iter; 1.0030x vs baseline; 1.0030x over previous
import jax
import jax.numpy as jnp
from jax import lax
from jax.experimental import pallas as pl
from jax.experimental.pallas import tpu as pltpu

N_DEV = 16
K_SUB = 2


def kernel(x, w_mat):
    m, k = x.shape
    _, n = w_mat.shape
    m_per = m // N_DEV
    nh = n // 2
    m_sub = m_per // K_SUB

    def body(x_ref, w_ref, out_ref,
             fsend_ref, bsend_ref, frecv_ref, brecv_ref,
             fsend_sems, bsend_sems, frecv_sems, brecv_sems):
        p = lax.axis_index("i")
        right = lax.rem(p + 1, N_DEV)
        left = lax.rem(p - 1 + N_DEV, N_DEV)

        def sub_dot(c, ks, lo):
            xs = x_ref[pl.ds(c * m_per + ks * m_sub, m_sub), :]
            return jax.lax.dot_general(
                xs, w_ref[:, pl.ds(lo, nh)],
                (((1,), (0,)), ((), ())),
                preferred_element_type=jnp.float32,
            )

        def make_rdma(s, ks, fwd):
            slot = s % 2
            send_ref = fsend_ref if fwd else bsend_ref
            recv_ref = frecv_ref if fwd else brecv_ref
            send_sems = fsend_sems if fwd else bsend_sems
            recv_sems = frecv_sems if fwd else brecv_sems
            return pltpu.make_async_remote_copy(
                src_ref=send_ref.at[slot, pl.ds(ks * m_sub, m_sub), :],
                dst_ref=recv_ref.at[s, pl.ds(ks * m_sub, m_sub), :],
                send_sem=send_sems.at[s, ks],
                recv_sem=recv_sems.at[s, ks],
                device_id=(right if fwd else left,),
                device_id_type=pl.DeviceIdType.MESH,
            )

        rows = lambda ks: pl.ds(ks * m_sub, m_sub)

        barrier_sem = pltpu.get_barrier_semaphore()
        for nbr in (left, right):
            pl.semaphore_signal(
                barrier_sem, inc=1,
                device_id=(nbr,), device_id_type=pl.DeviceIdType.MESH,
            )
        pl.semaphore_wait(barrier_sem, 2)

        fc0 = lax.rem(p - 1 + N_DEV, N_DEV)
        bc0 = lax.rem(p + 1, N_DEV)
        for ks in range(K_SUB):
            fsend_ref[0, rows(ks), :] = sub_dot(fc0, ks, 0).astype(jnp.bfloat16)
            make_rdma(0, ks, True).start()
            bsend_ref[0, rows(ks), :] = sub_dot(bc0, ks, nh).astype(jnp.bfloat16)
            make_rdma(0, ks, False).start()

        fc1 = lax.rem(p - 2 + N_DEV, N_DEV)
        bc1 = lax.rem(p + 2, N_DEV)
        fdots = [sub_dot(fc1, ks, 0) for ks in range(K_SUB)]
        bdots = [sub_dot(bc1, ks, nh) for ks in range(K_SUB)]

        for s in range(1, N_DEV - 1):
            slot = s % 2
            nfc = lax.rem(p - s - 2 + 2 * N_DEV, N_DEV) if s < N_DEV - 2 else p
            nbc = lax.rem(p + s + 2, N_DEV) if s < N_DEV - 2 else p
            nfdots = [sub_dot(nfc, ks, 0) for ks in range(K_SUB)]
            nbdots = [sub_dot(nbc, ks, nh) for ks in range(K_SUB)]
            for ks in range(K_SUB):
                if s >= 2:
                    make_rdma(s - 2, ks, True).wait_send()
                    make_rdma(s - 2, ks, False).wait_send()
                make_rdma(s - 1, ks, True).wait_recv()
                fsend_ref[slot, rows(ks), :] = (
                    frecv_ref[s - 1, rows(ks), :].astype(jnp.float32) + fdots[ks]
                ).astype(jnp.bfloat16)
                make_rdma(s, ks, True).start()

                make_rdma(s - 1, ks, False).wait_recv()
                bsend_ref[slot, rows(ks), :] = (
                    brecv_ref[s - 1, rows(ks), :].astype(jnp.float32) + bdots[ks]
                ).astype(jnp.bfloat16)
                make_rdma(s, ks, False).start()
            fdots, bdots = nfdots, nbdots

        for ks in range(K_SUB):
            make_rdma(N_DEV - 2, ks, True).wait_recv()
            out_ref[rows(ks), pl.ds(0, nh)] = (
                frecv_ref[N_DEV - 2, rows(ks), :].astype(jnp.float32) + fdots[ks]
            )
            make_rdma(N_DEV - 2, ks, False).wait_recv()
            out_ref[rows(ks), pl.ds(nh, nh)] = (
                brecv_ref[N_DEV - 2, rows(ks), :].astype(jnp.float32) + bdots[ks]
            )

        for s in (N_DEV - 3, N_DEV - 2):
            for ks in range(K_SUB):
                make_rdma(s, ks, True).wait_send()
                make_rdma(s, ks, False).wait_send()

    return pl.pallas_call(
        body,
        out_shape=jax.ShapeDtypeStruct((m_per, n), jnp.float32),
        in_specs=[
            pl.BlockSpec(memory_space=pltpu.VMEM),
            pl.BlockSpec(memory_space=pltpu.VMEM),
        ],
        out_specs=pl.BlockSpec(memory_space=pltpu.VMEM),
        scratch_shapes=[
            pltpu.VMEM((2, m_per, nh), jnp.bfloat16),
            pltpu.VMEM((2, m_per, nh), jnp.bfloat16),
            pltpu.VMEM((N_DEV - 1, m_per, nh), jnp.bfloat16),
            pltpu.VMEM((N_DEV - 1, m_per, nh), jnp.bfloat16),
            pltpu.SemaphoreType.DMA((N_DEV - 1, K_SUB)),
            pltpu.SemaphoreType.DMA((N_DEV - 1, K_SUB)),
            pltpu.SemaphoreType.DMA((N_DEV - 1, K_SUB)),
            pltpu.SemaphoreType.DMA((N_DEV - 1, K_SUB)),
        ],
        compiler_params=pltpu.CompilerParams(collective_id=0),
    )(x, w_mat)
